# baseline (speedup 1.0000x reference)
import jax
import jax.numpy as jnp
from jax.experimental import pallas as pl
from jax.experimental.pallas import tpu as pltpu

B, V, H, K, D = 1024, 10000, 512, 100, 128

VP = 10240
BB = 128
NBB = B // BB


def _x_copy(x_hbm, xbuf, sem_x, block, slot):
    return pltpu.make_async_copy(
        x_hbm.at[pl.ds(block * BB, BB), :], xbuf.at[slot], sem_x.at[slot])


def _fused_kernel(x_hbm, w1_hbm, b1_ref, wmu_ref, bmu_ref, wls_ref, bls_ref,
                  dm_ref, noise_ref, emb_hbm, te_ref,
                  rec_ref, kld_ref, me_ref, dp_ref,
                  xbuf, w1_vmem, emb_vmem, beta_vmem, sem_x, sem_w1, sem_emb):
    i = pl.program_id(0)
    slot = jax.lax.rem(i, 2)
    nslot = jax.lax.rem(i + 1, 2)

    @pl.when(i == 0)
    def _():
        _x_copy(x_hbm, xbuf, sem_x, 0, 0).start()
        w1_copy = pltpu.make_async_copy(w1_hbm, w1_vmem, sem_w1)
        w1_copy.start()
        emb_copy = pltpu.make_async_copy(emb_hbm, emb_vmem, sem_emb)
        emb_copy.start()
        emb_copy.wait()
        te = te_ref[...]
        s = jax.lax.dot_general(
            emb_vmem[...], te, (((1,), (1,)), ((), ())),
            preferred_element_type=jnp.float32)
        m = jnp.max(s, axis=0, keepdims=True)
        e = jnp.exp(s - m)
        den = jnp.sum(e, axis=0, keepdims=True)
        beta_vmem[:V, :] = (e / den).astype(jnp.bfloat16)
        beta_vmem[V:, :] = jnp.zeros((VP - V, K), jnp.bfloat16)
        nrm = jnp.sqrt(jnp.sum(te * te, axis=-1, keepdims=True))
        nt = te / (nrm + 1e-12)
        cosine = jnp.abs(jax.lax.dot_general(
            nt, nt, (((1,), (1,)), ((), ())),
            preferred_element_type=jnp.float32))
        cmean = jnp.mean(cosine)
        cvar = jnp.mean((cosine - cmean) ** 2)
        dp_ref[...] = (cmean - cvar).reshape(1, 1)
        w1_copy.wait()

    @pl.when(i + 1 < NBB)
    def _():
        _x_copy(x_hbm, xbuf, sem_x, i + 1, nslot).start()

    _x_copy(x_hbm, xbuf, sem_x, i, slot).wait()
    xb = xbuf[slot]

    h = jnp.dot(xb, w1_vmem[...], preferred_element_type=jnp.float32)
    h = jax.nn.softplus(h + b1_ref[...]) * dm_ref[...]
    mu = jnp.dot(h, wmu_ref[...], preferred_element_type=jnp.float32) + bmu_ref[...]
    ls = jnp.dot(h, wls_ref[...], preferred_element_type=jnp.float32) + bls_ref[...]
    kld = -0.5 * jnp.sum(1.0 + ls - mu * mu - jnp.exp(ls), axis=-1, keepdims=True)
    z = jax.nn.softmax(noise_ref[...] * jnp.exp(0.5 * ls) + mu, axis=-1)
    logits = jax.lax.dot_general(
        z.astype(jnp.bfloat16), beta_vmem[...], (((1,), (1,)), ((), ())),
        preferred_element_type=jnp.float32)
    rec = -jnp.sum(jnp.log(logits + 1e-10) * xb.astype(jnp.float32),
                   axis=-1, keepdims=True)
    rec_ref[...] = rec
    kld_ref[...] = kld
    me_ref[...] = rec + kld


@jax.jit
def kernel(x, W1, b1, Wmu, bmu, Wls, bls, emb, topic_emb, drop_mask, noise):
    xp = jnp.pad(x.astype(jnp.bfloat16), ((0, 0), (0, VP - V)))
    w1p = jnp.pad(W1.astype(jnp.bfloat16), ((0, VP - V), (0, 0)))
    rec, kld, me, dp = pl.pallas_call(
        _fused_kernel,
        grid=(NBB,),
        in_specs=[
            pl.BlockSpec(memory_space=pl.ANY),
            pl.BlockSpec(memory_space=pl.ANY),
            pl.BlockSpec((1, H), lambda i: (0, 0)),
            pl.BlockSpec((H, K), lambda i: (0, 0)),
            pl.BlockSpec((1, K), lambda i: (0, 0)),
            pl.BlockSpec((H, K), lambda i: (0, 0)),
            pl.BlockSpec((1, K), lambda i: (0, 0)),
            pl.BlockSpec((BB, H), lambda i: (i, 0)),
            pl.BlockSpec((BB, K), lambda i: (i, 0)),
            pl.BlockSpec(memory_space=pl.ANY),
            pl.BlockSpec((K, D), lambda i: (0, 0)),
        ],
        out_specs=[
            pl.BlockSpec((BB, 1), lambda i: (i, 0)),
            pl.BlockSpec((BB, 1), lambda i: (i, 0)),
            pl.BlockSpec((BB, 1), lambda i: (i, 0)),
            pl.BlockSpec((1, 1), lambda i: (0, 0)),
        ],
        out_shape=[
            jax.ShapeDtypeStruct((B, 1), jnp.float32),
            jax.ShapeDtypeStruct((B, 1), jnp.float32),
            jax.ShapeDtypeStruct((B, 1), jnp.float32),
            jax.ShapeDtypeStruct((1, 1), jnp.float32),
        ],
        scratch_shapes=[
            pltpu.VMEM((2, BB, VP), jnp.bfloat16),
            pltpu.VMEM((VP, H), jnp.bfloat16),
            pltpu.VMEM((V, D), jnp.float32),
            pltpu.VMEM((VP, K), jnp.bfloat16),
            pltpu.SemaphoreType.DMA((2,)),
            pltpu.SemaphoreType.DMA,
            pltpu.SemaphoreType.DMA,
        ],
    )(xp, w1p, b1.reshape(1, H), Wmu, bmu.reshape(1, K), Wls,
      bls.reshape(1, K), drop_mask, noise, emb, topic_emb)

    rec = rec.reshape(B)
    kld = kld.reshape(B)
    me = me.reshape(B)
    ppenalty = jnp.zeros((3,), dtype=jnp.float32)
    loss = me + jnp.sum(ppenalty[:2])
    return loss, me, rec, kld, ppenalty, dp.reshape(())


# speedup vs baseline: 1.5044x; 1.5044x over previous
import jax
import jax.numpy as jnp
from jax.experimental import pallas as pl
from jax.experimental.pallas import tpu as pltpu

B, V, H, K, D = 1024, 10000, 512, 100, 128

BB = 128
NBB = B // BB


def _x_copy(x_hbm, xbuf, sem_x, block, slot):
    return pltpu.make_async_copy(
        x_hbm.at[pl.ds(block * BB, BB), :], xbuf.at[slot], sem_x.at[slot])


def _fused_kernel(x_hbm, w1_hbm, b1_hbm, wmu_hbm, bmu_hbm, wls_hbm, bls_hbm,
                  dm_hbm, noise_hbm, emb_hbm, te_hbm,
                  rec_ref, kld_ref, me_ref, dp_ref,
                  xbuf, w1_vmem, emb_vmem, beta_vmem, small_vmem, hd_vmem,
                  sem_x, sem_w1, sem_emb, sem_s):
    i = pl.program_id(0)
    slot = jax.lax.rem(i, 2)
    nslot = jax.lax.rem(i + 1, 2)

    @pl.when(i == 0)
    def _():
        _x_copy(x_hbm, xbuf, sem_x, 0, 0).start()
        w1_copy = pltpu.make_async_copy(w1_hbm, w1_vmem, sem_w1)
        w1_copy.start()
        emb_copy = pltpu.make_async_copy(emb_hbm, emb_vmem, sem_emb)
        emb_copy.start()
        copies = [
            pltpu.make_async_copy(dm_hbm, small_vmem.dm, sem_s.at[0]),
            pltpu.make_async_copy(noise_hbm, small_vmem.noise, sem_s.at[1]),
            pltpu.make_async_copy(wmu_hbm, small_vmem.wmu, sem_s.at[2]),
            pltpu.make_async_copy(wls_hbm, small_vmem.wls, sem_s.at[3]),
            pltpu.make_async_copy(b1_hbm, small_vmem.b1, sem_s.at[4]),
            pltpu.make_async_copy(bmu_hbm, small_vmem.bmu, sem_s.at[5]),
            pltpu.make_async_copy(bls_hbm, small_vmem.bls, sem_s.at[6]),
            pltpu.make_async_copy(te_hbm, small_vmem.te, sem_s.at[7]),
        ]
        for c in copies:
            c.start()
        emb_copy.wait()
        for c in copies:
            c.wait()
        te = small_vmem.te[...]
        s = jax.lax.dot_general(
            emb_vmem[...], te, (((1,), (1,)), ((), ())),
            preferred_element_type=jnp.float32)
        m = jnp.max(s, axis=0, keepdims=True)
        e = jnp.exp(s - m)
        den = jnp.sum(e, axis=0, keepdims=True)
        beta_vmem[...] = (e / den).astype(jnp.bfloat16)
        nrm = jnp.sqrt(jnp.sum(te * te, axis=-1, keepdims=True))
        nt = te / (nrm + 1e-12)
        cosine = jnp.abs(jax.lax.dot_general(
            nt, nt, (((1,), (1,)), ((), ())),
            preferred_element_type=jnp.float32))
        cmean = jnp.mean(cosine)
        cvar = jnp.mean((cosine - cmean) ** 2)
        dp_ref[...] = (cmean - cvar).reshape(1, 1)
        w1_copy.wait()

    @pl.when(i + 1 < NBB)
    def _():
        _x_copy(x_hbm, xbuf, sem_x, i + 1, nslot).start()

    _x_copy(x_hbm, xbuf, sem_x, i, slot).wait()
    xb = xbuf[slot]
    row0 = i * BB

    h = jnp.dot(xb, w1_vmem[...], preferred_element_type=jnp.float32)
    h = jax.nn.softplus(h + small_vmem.b1[...]) * small_vmem.dm[pl.ds(row0, BB), :]
    mu = jnp.dot(h, small_vmem.wmu[...],
                 preferred_element_type=jnp.float32) + small_vmem.bmu[...]
    ls = jnp.dot(h, small_vmem.wls[...],
                 preferred_element_type=jnp.float32) + small_vmem.bls[...]
    kld = -0.5 * jnp.sum(1.0 + ls - mu * mu - jnp.exp(ls), axis=-1, keepdims=True)
    z = jax.nn.softmax(
        small_vmem.noise[pl.ds(row0, BB), :] * jnp.exp(0.5 * ls) + mu, axis=-1)
    logits = jax.lax.dot_general(
        z.astype(jnp.bfloat16), beta_vmem[...], (((1,), (1,)), ((), ())),
        preferred_element_type=jnp.float32)
    rec = -jnp.sum(jnp.log(logits + 1e-10) * xb, axis=-1, keepdims=True)
    rec_ref[...] = rec
    kld_ref[...] = kld
    me_ref[...] = rec + kld


class _Smalls:
    def __init__(self, dm, noise, wmu, wls, b1, bmu, bls, te):
        self.dm, self.noise, self.wmu, self.wls = dm, noise, wmu, wls
        self.b1, self.bmu, self.bls, self.te = b1, bmu, bls, te


def _kernel_wrapper(*refs):
    (x_hbm, w1_hbm, b1_hbm, wmu_hbm, bmu_hbm, wls_hbm, bls_hbm,
     dm_hbm, noise_hbm, emb_hbm, te_hbm,
     rec_ref, kld_ref, me_ref, dp_ref,
     xbuf, w1_vmem, emb_vmem, beta_vmem,
     dm_s, noise_s, wmu_s, wls_s, b1_s, bmu_s, bls_s, te_s,
     sem_x, sem_w1, sem_emb, sem_s) = refs
    smalls = _Smalls(dm_s, noise_s, wmu_s, wls_s, b1_s, bmu_s, bls_s, te_s)
    _fused_kernel(x_hbm, w1_hbm, b1_hbm, wmu_hbm, bmu_hbm, wls_hbm, bls_hbm,
                  dm_hbm, noise_hbm, emb_hbm, te_hbm,
                  rec_ref, kld_ref, me_ref, dp_ref,
                  xbuf, w1_vmem, emb_vmem, beta_vmem, smalls, None,
                  sem_x, sem_w1, sem_emb, sem_s)


@jax.jit
def kernel(x, W1, b1, Wmu, bmu, Wls, bls, emb, topic_emb, drop_mask, noise):
    rec, kld, me, dp = pl.pallas_call(
        _kernel_wrapper,
        grid=(NBB,),
        in_specs=[pl.BlockSpec(memory_space=pl.ANY)] * 11,
        out_specs=[
            pl.BlockSpec((BB, 1), lambda i: (i, 0)),
            pl.BlockSpec((BB, 1), lambda i: (i, 0)),
            pl.BlockSpec((BB, 1), lambda i: (i, 0)),
            pl.BlockSpec((1, 1), lambda i: (0, 0)),
        ],
        out_shape=[
            jax.ShapeDtypeStruct((B, 1), jnp.float32),
            jax.ShapeDtypeStruct((B, 1), jnp.float32),
            jax.ShapeDtypeStruct((B, 1), jnp.float32),
            jax.ShapeDtypeStruct((1, 1), jnp.float32),
        ],
        scratch_shapes=[
            pltpu.VMEM((2, BB, V), jnp.float32),
            pltpu.VMEM((V, H), jnp.float32),
            pltpu.VMEM((V, D), jnp.float32),
            pltpu.VMEM((V, K), jnp.bfloat16),
            pltpu.VMEM((B, H), jnp.float32),
            pltpu.VMEM((B, K), jnp.float32),
            pltpu.VMEM((H, K), jnp.float32),
            pltpu.VMEM((H, K), jnp.float32),
            pltpu.VMEM((1, H), jnp.float32),
            pltpu.VMEM((1, K), jnp.float32),
            pltpu.VMEM((1, K), jnp.float32),
            pltpu.VMEM((K, D), jnp.float32),
            pltpu.SemaphoreType.DMA((2,)),
            pltpu.SemaphoreType.DMA,
            pltpu.SemaphoreType.DMA,
            pltpu.SemaphoreType.DMA((8,)),
        ],
    )(x, W1, b1.reshape(1, H), Wmu, bmu.reshape(1, K), Wls,
      bls.reshape(1, K), drop_mask, noise, emb, topic_emb)

    rec = rec.reshape(B)
    kld = kld.reshape(B)
    me = me.reshape(B)
    ppenalty = jnp.zeros((3,), dtype=jnp.float32)
    loss = me + jnp.sum(ppenalty[:2])
    return loss, me, rec, kld, ppenalty, dp.reshape(())
